# baseline (speedup 1.0000x reference)
import jax
import jax.numpy as jnp
from jax.experimental import pallas as pl
from jax.experimental.pallas import tpu as pltpu

_B, _S, _H, _V, _E, _K = 1, 2048, 768, 32000, 8, 2
_NH = 4
_DH = _H // _NH
_I = _H * 2
_CH = _H // 4

_F32 = jnp.float32
_PREC = jax.lax.Precision.DEFAULT
_PREC_LO = jax.lax.Precision.DEFAULT

_G = 8
_QT = 1024
_TT = 512
_VT = 640


def _dot(a, b):
    return jax.lax.dot_general(a, b, (((1,), (0,)), ((), ())),
                               precision=_PREC, preferred_element_type=_F32)


def _dot_t(a, b):
    return jax.lax.dot_general(a, b, (((1,), (1,)), ((), ())),
                               precision=_PREC, preferred_element_type=_F32)


def _dot_t_lo(a, b):
    return jax.lax.dot_general(a, b, (((1,), (1,)), ((), ())),
                               precision=_PREC_LO, preferred_element_type=_F32)


def _gelu_exact(x):
    z = -x * 0.707106769
    ax = jnp.abs(z)
    w = z * z
    pe = w * 7.85386146e-05 + -0.000801019371
    pe = pe * w + 0.00518832775
    pe = pe * w + -0.0268538129
    pe = pe * w + 0.112835854
    pe = pe * w + -0.37612626
    pe = pe * w + 1.12837911
    small = 1.0 - z * pe
    rw = 1.0 / w
    ra = 1.0 / ax
    e = jnp.exp(-w)
    p1 = rw * 0.0232682 + -0.138703942
    p1 = p1 * rw + 0.368742466
    p1 = p1 * rw + -0.582473278
    p1 = p1 * rw + 0.621000469
    p1 = p1 * rw + -0.494451523
    p1 = p1 * rw + 0.340488
    p1 = p1 * rw + -0.274112701
    p1 = p1 * rw + 0.563825965
    p2 = rw * -10.477664 + 12.9772
    p2 = p2 * rw + -7.49551868
    p2 = p2 * rw + 2.92101908
    p2 = p2 * rw + -1.01526523
    p2 = p2 * rw + 0.42184633
    p2 = p2 * rw + -0.282076746
    p2 = p2 * rw + 0.564189494
    big = (e * ra) * jnp.where(ax < 2.0, p1, p2)
    big = jnp.where(-w < -88.7228394, 0.0, big)
    big = jnp.where(z < 0.0, 2.0 - big, big)
    erfc = jnp.where(ax < 1.0, small, big)
    return (x * 0.5) * erfc


def _ln_host(x, g, b):
    m = jnp.mean(x, axis=-1, keepdims=True)
    v = jnp.mean((x - m) ** 2, axis=-1, keepdims=True)
    return (x - m) / jnp.sqrt(v + 1e-5) * g + b


def _embed_body(ids_ref, *refs):
    del ids_ref
    tok_rows = [refs[j][0] for j in range(_G)]
    pos_ref, out_ref = refs[_G], refs[_G + 1]
    out_ref[...] = jnp.concatenate(tok_rows, axis=0) + pos_ref[...]


def _embed(ids, tok_emb, pos_emb):
    def _row_spec(j):
        return pl.BlockSpec((1, 1, _H),
                            lambda i, ids_ref, j=j: (ids_ref[i * _G + j], 0, 0))

    grid_spec = pltpu.PrefetchScalarGridSpec(
        num_scalar_prefetch=1,
        grid=(_S // _G,),
        in_specs=[_row_spec(j) for j in range(_G)]
        + [pl.BlockSpec((_G, _H), lambda i, ids_ref: (i, 0))],
        out_specs=pl.BlockSpec((_G, _H), lambda i, ids_ref: (i, 0)),
    )
    tok3 = tok_emb.reshape(_V, 1, _H)
    return pl.pallas_call(
        _embed_body,
        grid_spec=grid_spec,
        out_shape=jax.ShapeDtypeStruct((_S, _H), _F32),
        compiler_params=pltpu.CompilerParams(
            dimension_semantics=("arbitrary",),
            vmem_limit_bytes=48 * 1024 * 1024,
        ),
    )(ids, *([tok3] * _G), pos_emb)


def _qkv_body(x_ref, w_ref, wb_ref, out_ref):
    out_ref[...] = _dot_t(x_ref[...], w_ref[...]) + wb_ref[...]


def _qkv(hn, in_proj_w, in_proj_b):
    nt = 8
    ts = _S // nt
    return pl.pallas_call(
        _qkv_body,
        grid=(nt,),
        in_specs=[
            pl.BlockSpec((ts, _H), lambda i: (i, 0)),
            pl.BlockSpec((3 * _H, _H), lambda i: (0, 0)),
            pl.BlockSpec((1, 3 * _H), lambda i: (0, 0)),
        ],
        out_specs=pl.BlockSpec((ts, 3 * _H), lambda i: (i, 0)),
        out_shape=jax.ShapeDtypeStruct((_S, 3 * _H), _F32),
        compiler_params=pltpu.CompilerParams(
            dimension_semantics=("parallel",),
            vmem_limit_bytes=60 * 1024 * 1024,
        ),
    )(hn, in_proj_w, in_proj_b.reshape(1, 3 * _H))


_KV = 1024


def _attn_body(q_ref, k_ref, v_ref, out_ref):
    q = q_ref[0]
    k = k_ref[0]
    v = v_ref[0]
    s = _dot_t(q, k) * (1.0 / (_DH ** 0.5))
    s0 = s[:, :_KV]
    s1 = s[:, _KV:]
    m0 = jnp.max(s0, axis=-1, keepdims=True)
    p0 = jnp.exp(s0 - m0)
    l0 = jnp.sum(p0, axis=-1, keepdims=True)
    acc0 = _dot(p0, v[:_KV])
    o0 = acc0 * (1.0 / l0)
    mc = jnp.max(s1, axis=-1, keepdims=True)
    m1 = jnp.maximum(m0, mc)
    dm = jnp.where(m0 == m1, 0.0, m0 - m1)
    edm = jnp.exp(dm)
    p1 = jnp.exp(s1 - m1)
    l1 = edm * l0 + jnp.sum(p1, axis=-1, keepdims=True)
    acc1 = _dot(p1, v[_KV:]) + (edm * l0) * o0
    out_ref[0] = acc1 * (1.0 / l1)


def _attention(qkv):
    qkv3 = qkv.reshape(_S, 12, _DH).transpose(1, 0, 2)
    nq = _S // _QT
    out = pl.pallas_call(
        _attn_body,
        grid=(_NH, nq),
        in_specs=[
            pl.BlockSpec((1, _QT, _DH), lambda h, i: (h, i, 0)),
            pl.BlockSpec((1, _S, _DH), lambda h, i: (4 + h, 0, 0)),
            pl.BlockSpec((1, _S, _DH), lambda h, i: (8 + h, 0, 0)),
        ],
        out_specs=pl.BlockSpec((1, _QT, _DH), lambda h, i: (h, i, 0)),
        out_shape=jax.ShapeDtypeStruct((_NH, _S, _DH), _F32),
        compiler_params=pltpu.CompilerParams(
            dimension_semantics=("arbitrary", "arbitrary"),
            vmem_limit_bytes=60 * 1024 * 1024,
        ),
    )(qkv3, qkv3, qkv3)
    return out.transpose(1, 0, 2).reshape(_S, _H)


def _proj_body(o_ref, h_ref, w_ref, b_ref, h2_ref):
    attn_out = _dot_t(o_ref[...], w_ref[...]) + b_ref[...]
    h2_ref[...] = h_ref[...] + attn_out


def _proj(attn_o, h1, out_proj_w, out_proj_b):
    full = lambda shp: pl.BlockSpec(shp, lambda: (0,) * len(shp))
    return pl.pallas_call(
        _proj_body,
        in_specs=[full((_S, _H)), full((_S, _H)), full((_H, _H)),
                  full((1, _H))],
        out_specs=full((_S, _H)),
        out_shape=jax.ShapeDtypeStruct((_S, _H), _F32),
        compiler_params=pltpu.CompilerParams(
            vmem_limit_bytes=60 * 1024 * 1024,
        ),
    )(attn_o, h1, out_proj_w, out_proj_b.reshape(1, _H))


def _router_body(hn2_ref, cw_ref, cb_ref, gw_ref, t_ref,
                 rw_ref, wf_ref, us_ref):
    hn2 = hn2_ref[...]
    ctx = _gelu_exact(_dot_t(hn2, cw_ref[...]) + cb_ref[...])
    logits = _dot_t(ctx, gw_ref[...]) / t_ref[0, 0]
    lm = jnp.max(logits, axis=-1, keepdims=True)
    p = jnp.exp(logits - lm)
    rw = p / jnp.sum(p, axis=-1, keepdims=True)
    rw_ref[...] = rw
    lane = jax.lax.broadcasted_iota(jnp.int32, (_S, _E), 1)
    v1 = jnp.max(rw, axis=-1, keepdims=True)
    i1 = jnp.min(jnp.where(rw >= v1, lane, _E), axis=-1, keepdims=True)
    masked = jnp.where(lane == i1, -jnp.inf, rw)
    v2 = jnp.max(masked, axis=-1, keepdims=True)
    i2 = jnp.min(jnp.where(masked >= v2, lane, _E), axis=-1, keepdims=True)
    wsum = v1 + v2
    wf_ref[...] = (jnp.where(lane == i1, v1, 0.0)
                   + jnp.where(lane == i2, v2, 0.0)) / wsum
    total = jnp.sum(rw)
    us_ref[...] = jnp.sum(rw, axis=0, keepdims=True) / total


def _router(hn2, ctx_w, ctx_b, gate_w, temperature):
    full = lambda shp: pl.BlockSpec(shp, lambda: (0,) * len(shp))
    return pl.pallas_call(
        _router_body,
        in_specs=[full((_S, _H)), full((_CH, _H)), full((1, _CH)),
                  full((_E, _CH)), full((1, 1))],
        out_specs=[full((_S, _E)), full((_S, _E)), full((1, _E))],
        out_shape=[
            jax.ShapeDtypeStruct((_S, _E), _F32),
            jax.ShapeDtypeStruct((_S, _E), _F32),
            jax.ShapeDtypeStruct((1, _E), _F32),
        ],
        compiler_params=pltpu.CompilerParams(
            vmem_limit_bytes=60 * 1024 * 1024,
        ),
    )(hn2, ctx_w, ctx_b.reshape(1, _CH), gate_w, temperature.reshape(1, 1))


def _moe_body(x_ref, wg_ref, wu_ref, wd_ref, wf_ref, out_ref):
    e = pl.program_id(1)
    x = x_ref[...]
    xg = _dot_t_lo(x, wg_ref[0])
    xu = _dot_t_lo(x, wu_ref[0])
    act = _gelu_exact(xg) * xu
    eo = _dot_t_lo(act, wd_ref[0])
    lane = jax.lax.broadcasted_iota(jnp.int32, (_TT, _E), 1)
    w = jnp.sum(jnp.where(lane == e, wf_ref[...], 0.0), axis=-1, keepdims=True)
    val = eo * w

    @pl.when(e == 0)
    def _init():
        out_ref[...] = val

    @pl.when(e != 0)
    def _acc():
        out_ref[...] = out_ref[...] + val


def _moe(hn2, wfull, Wg, Wu, Wd):
    nt = _S // _TT
    return pl.pallas_call(
        _moe_body,
        grid=(nt, _E),
        in_specs=[
            pl.BlockSpec((_TT, _H), lambda t, e: (t, 0)),
            pl.BlockSpec((1, _I, _H), lambda t, e: (e, 0, 0)),
            pl.BlockSpec((1, _I, _H), lambda t, e: (e, 0, 0)),
            pl.BlockSpec((1, _H, _I), lambda t, e: (e, 0, 0)),
            pl.BlockSpec((_TT, _E), lambda t, e: (t, 0)),
        ],
        out_specs=pl.BlockSpec((_TT, _H), lambda t, e: (t, 0)),
        out_shape=jax.ShapeDtypeStruct((_S, _H), _F32),
        compiler_params=pltpu.CompilerParams(
            dimension_semantics=("parallel", "arbitrary"),
            vmem_limit_bytes=60 * 1024 * 1024,
        ),
    )(hn2, Wg, Wu, Wd, wfull)


def _logits_body(h_ref, w_ref, b_ref, out_ref):
    out_ref[...] = _dot_t_lo(h_ref[...], w_ref[...]) + b_ref[...]


def _logits(h2, out_w, out_b):
    nv = _V // _VT
    return pl.pallas_call(
        _logits_body,
        grid=(nv,),
        in_specs=[
            pl.BlockSpec((_S, _H), lambda i: (0, 0)),
            pl.BlockSpec((_VT, _H), lambda i: (i, 0)),
            pl.BlockSpec((1, _VT), lambda i: (0, i)),
        ],
        out_specs=pl.BlockSpec((_S, _VT), lambda i: (0, i)),
        out_shape=jax.ShapeDtypeStruct((_S, _V), _F32),
        compiler_params=pltpu.CompilerParams(
            dimension_semantics=("parallel",),
            vmem_limit_bytes=60 * 1024 * 1024,
        ),
    )(h2, out_w, out_b.reshape(1, _V))


def kernel(input_ids, tok_emb, pos_emb, ln1_g, ln1_b, ln2_g, ln2_b,
           in_proj_w, in_proj_b, out_proj_w, out_proj_b,
           ctx_w, ctx_b, gate_w, temperature, Wg, Wu, Wd, out_w, out_b):
    ids = input_ids.reshape(_S)
    h1 = _embed(ids, tok_emb, pos_emb)
    hn1 = _ln_host(h1, ln1_g, ln1_b)
    qkv = _qkv(hn1, in_proj_w, in_proj_b)
    attn_o = _attention(qkv)
    h2 = _proj(attn_o, h1, out_proj_w, out_proj_b)
    hn2 = _ln_host(h2, ln2_g, ln2_b)
    rw, wfull, usage = _router(hn2, ctx_w, ctx_b, gate_w, temperature)
    moe = _moe(hn2, wfull, Wg, Wu, Wd)
    h3 = h2 + moe
    logits = _logits(h3, out_w, out_b)
    return (logits.reshape(_B, _S, _V), h3.reshape(_B, _S, _H),
            rw, usage.reshape(_E))
